# baseline (speedup 1.0000x reference)
import jax
import jax.numpy as jnp
from jax import lax
from jax.experimental import pallas as pl
from jax.experimental.pallas import tpu as pltpu
from jax.experimental.pallas import tpu_sc as plsc

N_NODES = 10000
N_PAD = 10112
D = 128
E = 320000
B = 16384

NC, NS = 2, 16
NW = NC * NS
CH = 80
NG = 2
GCH = 63
E_PER_W = NG * GCH * CH
E_PAD = NW * E_PER_W
ROWS_PER_TILE = N_PAD // NS

PAIRS_PER_W = B // NW
SCH = 128
NSCH = PAIRS_PER_W // SCH

_mesh = plsc.VectorSubcoreMesh(core_axis_name="c", subcore_axis_name="s")
_sc_params = pltpu.CompilerParams(use_tc_tiling_on_sc=False)
_sc_params_nl = pltpu.CompilerParams(use_tc_tiling_on_sc=False,
                                     needs_layout_passes=False)


def _agg_body(x_hbm, esrc_hbm, edst_hbm, feat_hbm, deg_hbm,
              feat_sh, deg_sh, idx_s, idx_d, b0, b1, b2, degz, ones_v,
              g0, g1, g2, s0, s1, s2):
    cid = lax.axis_index("c")
    sid = lax.axis_index("s")
    wid = cid * NS + sid

    zero = jnp.zeros((16,), jnp.float32)

    @pl.loop(0, CH)
    def _(r):
        for k in range(D // 16):
            b0[r, pl.ds(16 * k, 16)] = zero

    @pl.loop(0, ROWS_PER_TILE // 16)
    def _(r):
        degz[pl.ds(16 * r, 16)] = zero

    for k in range(CH // 16):
        ones_v[pl.ds(16 * k, 16)] = jnp.ones((16,), jnp.float32)

    row0 = sid * ROWS_PER_TILE
    for b in range(7):
        pltpu.sync_copy(b0, feat_sh.at[pl.ds(row0 + b * CH, CH)])
    pltpu.sync_copy(b0.at[pl.ds(0, 72)],
                    feat_sh.at[pl.ds(row0 + 7 * CH, 72)])
    pltpu.sync_copy(degz, deg_sh.at[pl.ds(row0, ROWS_PER_TILE)])
    plsc.subcore_barrier()

    def g(c, buf, sem):
        pltpu.async_copy(x_hbm.at[idx_s.at[c]], buf, sem)

    def wg(buf, sem):
        pltpu.make_async_copy(x_hbm.at[idx_s.at[0]], buf, sem).wait()

    def s(c, buf, sem):
        pltpu.async_copy(buf, feat_sh.at[idx_d.at[c]], sem, add=True)

    def ws(buf, sem):
        pltpu.make_async_copy(buf, feat_sh.at[idx_d.at[0]], sem).wait()

    def dg(c):
        pltpu.sync_copy(ones_v, deg_sh.at[idx_d.at[c]], add=True)

    for grp in range(NG):
        pltpu.sync_copy(esrc_hbm.at[wid, grp], idx_s)
        pltpu.sync_copy(edst_hbm.at[wid, grp], idx_d)

        g(0, b0, g0)
        g(1, b1, g1)
        wg(b0, g0); s(0, b0, s0); dg(0); g(2, b2, g2)
        wg(b1, g1); s(1, b1, s1); dg(1); ws(b0, s0); g(3, b0, g0)

        @pl.loop(0, (GCH - 6) // 3)
        def _(j):
            c0 = 3 * j + 2
            wg(b2, g2); s(c0, b2, s2); dg(c0); ws(b1, s1); g(c0 + 2, b1, g1)
            c1 = c0 + 1
            wg(b0, g0); s(c1, b0, s0); dg(c1); ws(b2, s2); g(c1 + 2, b2, g2)
            c2 = c0 + 2
            wg(b1, g1); s(c2, b1, s1); dg(c2); ws(b0, s0); g(c2 + 2, b0, g0)

        c = GCH - 4
        wg(b2, g2); s(c, b2, s2); dg(c); ws(b1, s1); g(c + 2, b1, g1)
        c = GCH - 3
        wg(b0, g0); s(c, b0, s0); dg(c); ws(b2, s2); g(c + 2, b2, g2)
        c = GCH - 2
        wg(b1, g1); s(c, b1, s1); dg(c)
        c = GCH - 1
        wg(b2, g2); s(c, b2, s2); dg(c)
        ws(b0, s0)
        ws(b1, s1)
        ws(b2, s2)

    plsc.subcore_barrier()
    pltpu.sync_copy(feat_sh.at[pl.ds(row0, ROWS_PER_TILE)],
                    feat_hbm.at[cid, pl.ds(row0, ROWS_PER_TILE)])
    pltpu.sync_copy(deg_sh.at[pl.ds(row0, ROWS_PER_TILE)],
                    deg_hbm.at[cid, pl.ds(row0, ROWS_PER_TILE)])


_agg_call = pl.kernel(
    _agg_body,
    out_type=[
        jax.ShapeDtypeStruct((NC, N_PAD, D), jnp.float32),
        jax.ShapeDtypeStruct((NC, N_PAD), jnp.float32),
    ],
    mesh=_mesh,
    scratch_types=[
        pltpu.VMEM_SHARED((N_PAD, D), jnp.float32),
        pltpu.VMEM_SHARED((N_PAD,), jnp.float32),
        pltpu.VMEM((GCH, CH), jnp.int32),
        pltpu.VMEM((GCH, CH), jnp.int32),
        pltpu.VMEM((CH, D), jnp.float32),
        pltpu.VMEM((CH, D), jnp.float32),
        pltpu.VMEM((CH, D), jnp.float32),
        pltpu.VMEM((ROWS_PER_TILE,), jnp.float32),
        pltpu.VMEM((CH,), jnp.float32),
        pltpu.SemaphoreType.DMA,
        pltpu.SemaphoreType.DMA,
        pltpu.SemaphoreType.DMA,
        pltpu.SemaphoreType.DMA,
        pltpu.SemaphoreType.DMA,
        pltpu.SemaphoreType.DMA,
    ],
    compiler_params=_sc_params,
)


RB = 1000


def _dense_body(feat_ref, deg_ref, x_ref, ws_ref, wn_ref, h_ref):
    a = feat_ref[0] + feat_ref[1]
    deg = deg_ref[:, 0:1] + deg_ref[:, 1:2]
    mean = a / jnp.maximum(deg, 1.0)
    h_ref[...] = jnp.maximum(
        jnp.dot(x_ref[...], ws_ref[...], preferred_element_type=jnp.float32)
        + jnp.dot(mean, wn_ref[...], preferred_element_type=jnp.float32),
        0.0)


_dense_call = pl.pallas_call(
    _dense_body,
    grid=(N_NODES // RB,),
    in_specs=[
        pl.BlockSpec((NC, RB, D), lambda i: (0, i, 0)),
        pl.BlockSpec((RB, NC), lambda i: (i, 0)),
        pl.BlockSpec((RB, D), lambda i: (i, 0)),
        pl.BlockSpec((D, D), lambda i: (0, 0)),
        pl.BlockSpec((D, D), lambda i: (0, 0)),
    ],
    out_specs=pl.BlockSpec((RB, D), lambda i: (i, 0)),
    out_shape=jax.ShapeDtypeStruct((N_NODES, D), jnp.float32),
)


def _pairgather_body(h_hbm, src_hbm, dst_hbm, nb_hbm,
                     us_hbm, vd_hbm, bsum_hbm,
                     idx_s, idx_d, hs, hd, nb_v, bias_v, sem0, sem1):
    cid = lax.axis_index("c")
    sid = lax.axis_index("s")
    wid = cid * NS + sid

    pltpu.sync_copy(src_hbm.at[wid], idx_s)
    pltpu.sync_copy(dst_hbm.at[wid], idx_d)
    pltpu.sync_copy(nb_hbm, nb_v)

    @pl.loop(0, NSCH)
    def _(c):
        cp0 = pltpu.async_copy(h_hbm.at[idx_s.at[c]], hs, sem0)
        cp1 = pltpu.async_copy(h_hbm.at[idx_d.at[c]], hd, sem1)
        for g in range(SCH // 16):
            sv = idx_s[c, pl.ds(16 * g, 16)]
            dv = idx_d[c, pl.ds(16 * g, 16)]
            bs = plsc.load_gather(nb_v, [sv])
            bd = plsc.load_gather(nb_v, [dv])
            bias_v[pl.ds(16 * g, 16)] = bs + bd
        cp0.wait()
        cp1.wait()
        base = wid * PAIRS_PER_W + c * SCH
        pltpu.sync_copy(hs, us_hbm.at[pl.ds(base, SCH)])
        pltpu.sync_copy(hd, vd_hbm.at[pl.ds(base, SCH)])
        pltpu.sync_copy(bias_v, bsum_hbm.at[pl.ds(base, SCH)])


_pairgather_call = pl.kernel(
    _pairgather_body,
    out_type=[
        jax.ShapeDtypeStruct((B, D), jnp.float32),
        jax.ShapeDtypeStruct((B, D), jnp.float32),
        jax.ShapeDtypeStruct((B,), jnp.float32),
    ],
    mesh=_mesh,
    scratch_types=[
        pltpu.VMEM((NSCH, SCH), jnp.int32),
        pltpu.VMEM((NSCH, SCH), jnp.int32),
        pltpu.VMEM((SCH, D), jnp.float32),
        pltpu.VMEM((SCH, D), jnp.float32),
        pltpu.VMEM((N_NODES,), jnp.float32),
        pltpu.VMEM((SCH,), jnp.float32),
        pltpu.SemaphoreType.DMA,
        pltpu.SemaphoreType.DMA,
    ],
    compiler_params=_sc_params_nl,
)


SB = 2048


def _score_body(us_ref, vd_ref, bsum_ref, out_ref):
    out_ref[...] = jnp.sum(us_ref[...] * vd_ref[...], axis=1) + bsum_ref[...]


_score_call = pl.pallas_call(
    _score_body,
    grid=(B // SB,),
    in_specs=[
        pl.BlockSpec((SB, D), lambda i: (i, 0)),
        pl.BlockSpec((SB, D), lambda i: (i, 0)),
        pl.BlockSpec((SB,), lambda i: (i,)),
    ],
    out_specs=pl.BlockSpec((SB,), lambda i: (i,)),
    out_shape=jax.ShapeDtypeStruct((B,), jnp.float32),
)


def kernel(x, edge_index, src, dst, W_self, W_neigh, node_biases):
    x = x.astype(jnp.float32)
    npad = E_PAD - E
    e_src = jnp.concatenate(
        [edge_index[0].astype(jnp.int32), jnp.zeros((npad,), jnp.int32)]
    ).reshape(NW, NG, GCH, CH)
    e_dst = jnp.concatenate(
        [edge_index[1].astype(jnp.int32),
         jnp.full((npad,), N_PAD - 1, jnp.int32)]
    ).reshape(NW, NG, GCH, CH)
    src3 = src.astype(jnp.int32).reshape(NW, NSCH, SCH)
    dst3 = dst.astype(jnp.int32).reshape(NW, NSCH, SCH)
    nb = node_biases[1:N_NODES + 1].astype(jnp.float32)

    feat, deg = _agg_call(x, e_src, e_dst)
    h = _dense_call(feat, deg.T, x, W_self, W_neigh)
    us, vd, bsum = _pairgather_call(h, src3, dst3, nb)
    return _score_call(us, vd, bsum)


# speedup vs baseline: 1.5148x; 1.5148x over previous
import jax
import jax.numpy as jnp
from jax import lax
from jax.experimental import pallas as pl
from jax.experimental.pallas import tpu as pltpu
from jax.experimental.pallas import tpu_sc as plsc

N_NODES = 10000
N_PAD = 10240
D = 128
E = 320000
B = 16384

NC, NS = 2, 16
NW = NC * NS
E_PER_W = E // NW
CH = 80
NCH = E_PER_W // CH
ROWS_PER_TILE = N_PAD // NS

PAIRS_PER_W = B // NW
SCH = 128
NSCH = PAIRS_PER_W // SCH

_mesh = plsc.VectorSubcoreMesh(core_axis_name="c", subcore_axis_name="s")
_sc_params = pltpu.CompilerParams(use_tc_tiling_on_sc=False)
_sc_params_nl = pltpu.CompilerParams(use_tc_tiling_on_sc=False,
                                     needs_layout_passes=False)


def _agg_body(x_hbm, esrc_hbm, edst_hbm, feat_hbm, deg_hbm,
              feat_sh, deg_sh, idx_s, idx_d, rows0, rows1, degz, ones_v,
              sem0, sem1):
    cid = lax.axis_index("c")
    sid = lax.axis_index("s")
    wid = cid * NS + sid

    zero = jnp.zeros((16,), jnp.float32)

    @pl.loop(0, CH)
    def _(r):
        for k in range(D // 16):
            rows0[r, pl.ds(16 * k, 16)] = zero

    @pl.loop(0, ROWS_PER_TILE // 16)
    def _(r):
        degz[pl.ds(16 * r, 16)] = zero

    for k in range(CH // 16):
        ones_v[pl.ds(16 * k, 16)] = jnp.ones((16,), jnp.float32)

    row0 = sid * ROWS_PER_TILE
    for b in range(ROWS_PER_TILE // CH):
        pltpu.sync_copy(rows0, feat_sh.at[pl.ds(row0 + b * CH, CH)])
    pltpu.sync_copy(degz, deg_sh.at[pl.ds(row0, ROWS_PER_TILE)])
    plsc.subcore_barrier()

    pltpu.sync_copy(esrc_hbm.at[wid], idx_s)
    pltpu.sync_copy(edst_hbm.at[wid], idx_d)

    def start(c, buf, sem):
        pltpu.async_copy(x_hbm.at[idx_s.at[c]], buf, sem)

    def wait(buf, sem):
        pltpu.make_async_copy(x_hbm.at[idx_s.at[0]], buf, sem).wait()

    def scat(c, buf):
        pltpu.sync_copy(buf, feat_sh.at[idx_d.at[c]], add=True)
        pltpu.sync_copy(ones_v, deg_sh.at[idx_d.at[c]], add=True)

    start(0, rows0, sem0)

    @pl.loop(0, NCH // 2)
    def _(j):
        c0 = 2 * j
        start(c0 + 1, rows1, sem1)
        wait(rows0, sem0)
        scat(c0, rows0)
        start(c0 + 2, rows0, sem0)
        wait(rows1, sem1)
        scat(c0 + 1, rows1)

    wait(rows0, sem0)
    scat(NCH - 1, rows0)

    plsc.subcore_barrier()
    pltpu.sync_copy(feat_sh.at[pl.ds(row0, ROWS_PER_TILE)],
                    feat_hbm.at[cid, pl.ds(row0, ROWS_PER_TILE)])
    pltpu.sync_copy(deg_sh.at[pl.ds(row0, ROWS_PER_TILE)],
                    deg_hbm.at[cid, pl.ds(row0, ROWS_PER_TILE)])


_agg_call = pl.kernel(
    _agg_body,
    out_type=[
        jax.ShapeDtypeStruct((NC, N_PAD, D), jnp.float32),
        jax.ShapeDtypeStruct((NC, N_PAD), jnp.float32),
    ],
    mesh=_mesh,
    scratch_types=[
        pltpu.VMEM_SHARED((N_PAD, D), jnp.float32),
        pltpu.VMEM_SHARED((N_PAD,), jnp.float32),
        pltpu.VMEM((NCH, CH), jnp.int32),
        pltpu.VMEM((NCH, CH), jnp.int32),
        pltpu.VMEM((CH, D), jnp.float32),
        pltpu.VMEM((CH, D), jnp.float32),
        pltpu.VMEM((ROWS_PER_TILE,), jnp.float32),
        pltpu.VMEM((CH,), jnp.float32),
        pltpu.SemaphoreType.DMA,
        pltpu.SemaphoreType.DMA,
    ],
    compiler_params=_sc_params,
)


RB = 1000


def _dense_body(feat_ref, deg_ref, x_ref, ws_ref, wn_ref, h_ref):
    a = feat_ref[0] + feat_ref[1]
    deg = deg_ref[:, 0:1] + deg_ref[:, 1:2]
    mean = a / jnp.maximum(deg, 1.0)
    h_ref[...] = jnp.maximum(
        jnp.dot(x_ref[...], ws_ref[...], preferred_element_type=jnp.float32)
        + jnp.dot(mean, wn_ref[...], preferred_element_type=jnp.float32),
        0.0)


_dense_call = pl.pallas_call(
    _dense_body,
    grid=(N_NODES // RB,),
    in_specs=[
        pl.BlockSpec((NC, RB, D), lambda i: (0, i, 0)),
        pl.BlockSpec((RB, NC), lambda i: (i, 0)),
        pl.BlockSpec((RB, D), lambda i: (i, 0)),
        pl.BlockSpec((D, D), lambda i: (0, 0)),
        pl.BlockSpec((D, D), lambda i: (0, 0)),
    ],
    out_specs=pl.BlockSpec((RB, D), lambda i: (i, 0)),
    out_shape=jax.ShapeDtypeStruct((N_NODES, D), jnp.float32),
)


def _pairgather_body(h_hbm, src_hbm, dst_hbm, nb_hbm,
                     us_hbm, vd_hbm, bsum_hbm,
                     idx_s, idx_d, hs, hd, nb_v, bias_v, sem0, sem1):
    cid = lax.axis_index("c")
    sid = lax.axis_index("s")
    wid = cid * NS + sid

    pltpu.sync_copy(src_hbm.at[wid], idx_s)
    pltpu.sync_copy(dst_hbm.at[wid], idx_d)
    pltpu.sync_copy(nb_hbm, nb_v)

    @pl.loop(0, NSCH)
    def _(c):
        cp0 = pltpu.async_copy(h_hbm.at[idx_s.at[c]], hs, sem0)
        cp1 = pltpu.async_copy(h_hbm.at[idx_d.at[c]], hd, sem1)
        for g in range(SCH // 16):
            sv = idx_s[c, pl.ds(16 * g, 16)]
            dv = idx_d[c, pl.ds(16 * g, 16)]
            bs = plsc.load_gather(nb_v, [sv])
            bd = plsc.load_gather(nb_v, [dv])
            bias_v[pl.ds(16 * g, 16)] = bs + bd
        cp0.wait()
        cp1.wait()
        base = wid * PAIRS_PER_W + c * SCH
        pltpu.sync_copy(hs, us_hbm.at[pl.ds(base, SCH)])
        pltpu.sync_copy(hd, vd_hbm.at[pl.ds(base, SCH)])
        pltpu.sync_copy(bias_v, bsum_hbm.at[pl.ds(base, SCH)])


_pairgather_call = pl.kernel(
    _pairgather_body,
    out_type=[
        jax.ShapeDtypeStruct((B, D), jnp.float32),
        jax.ShapeDtypeStruct((B, D), jnp.float32),
        jax.ShapeDtypeStruct((B,), jnp.float32),
    ],
    mesh=_mesh,
    scratch_types=[
        pltpu.VMEM((NSCH, SCH), jnp.int32),
        pltpu.VMEM((NSCH, SCH), jnp.int32),
        pltpu.VMEM((SCH, D), jnp.float32),
        pltpu.VMEM((SCH, D), jnp.float32),
        pltpu.VMEM((N_NODES,), jnp.float32),
        pltpu.VMEM((SCH,), jnp.float32),
        pltpu.SemaphoreType.DMA,
        pltpu.SemaphoreType.DMA,
    ],
    compiler_params=_sc_params_nl,
)


SB = 2048


def _score_body(us_ref, vd_ref, bsum_ref, out_ref):
    out_ref[...] = jnp.sum(us_ref[...] * vd_ref[...], axis=1) + bsum_ref[...]


_score_call = pl.pallas_call(
    _score_body,
    grid=(B // SB,),
    in_specs=[
        pl.BlockSpec((SB, D), lambda i: (i, 0)),
        pl.BlockSpec((SB, D), lambda i: (i, 0)),
        pl.BlockSpec((SB,), lambda i: (i,)),
    ],
    out_specs=pl.BlockSpec((SB,), lambda i: (i,)),
    out_shape=jax.ShapeDtypeStruct((B,), jnp.float32),
)


def kernel(x, edge_index, src, dst, W_self, W_neigh, node_biases):
    x = x.astype(jnp.float32)
    e_src = edge_index[0].astype(jnp.int32).reshape(NW, NCH, CH)
    e_dst = edge_index[1].astype(jnp.int32).reshape(NW, NCH, CH)
    src3 = src.astype(jnp.int32).reshape(NW, NSCH, SCH)
    dst3 = dst.astype(jnp.int32).reshape(NW, NSCH, SCH)
    nb = node_biases[1:N_NODES + 1].astype(jnp.float32)

    feat, deg = _agg_call(x, e_src, e_dst)
    h = _dense_call(feat, deg.T, x, W_self, W_neigh)
    us, vd, bsum = _pairgather_call(h, src3, dst3, nb)
    return _score_call(us, vd, bsum)


# speedup vs baseline: 1.5365x; 1.0143x over previous
import jax
import jax.numpy as jnp
from jax import lax
from jax.experimental import pallas as pl
from jax.experimental.pallas import tpu as pltpu
from jax.experimental.pallas import tpu_sc as plsc

N_NODES = 10000
N_PAD = 10240
D = 128
E = 320000
B = 16384

NC, NS = 2, 16
NW = NC * NS
E_PER_W = E // NW
CH = 80
NCH = E_PER_W // CH
ROWS_PER_TILE = N_PAD // NS

PAIRS_PER_W = B // NW
SCH = 128
NSCH = PAIRS_PER_W // SCH

_mesh = plsc.VectorSubcoreMesh(core_axis_name="c", subcore_axis_name="s")
_sc_params = pltpu.CompilerParams(use_tc_tiling_on_sc=False)
_sc_params_nl = pltpu.CompilerParams(use_tc_tiling_on_sc=False,
                                     needs_layout_passes=False)


def _agg_body(x_hbm, esrc_hbm, edst_hbm, feat_hbm, deg_hbm,
              feat_sh, deg_sh, idx_s, idx_d, rows0, rows1, degz, ones_v,
              sem0, sem1, sem_d):
    cid = lax.axis_index("c")
    sid = lax.axis_index("s")
    wid = cid * NS + sid

    zero = jnp.zeros((16,), jnp.float32)

    @pl.loop(0, CH)
    def _(r):
        for k in range(D // 16):
            rows0[r, pl.ds(16 * k, 16)] = zero

    @pl.loop(0, ROWS_PER_TILE // 16)
    def _(r):
        degz[pl.ds(16 * r, 16)] = zero

    for k in range(CH // 16):
        ones_v[pl.ds(16 * k, 16)] = jnp.ones((16,), jnp.float32)

    row0 = sid * ROWS_PER_TILE
    for b in range(ROWS_PER_TILE // CH):
        pltpu.sync_copy(rows0, feat_sh.at[pl.ds(row0 + b * CH, CH)])
    pltpu.sync_copy(degz, deg_sh.at[pl.ds(row0, ROWS_PER_TILE)])
    plsc.subcore_barrier()

    pltpu.sync_copy(esrc_hbm.at[wid], idx_s)
    pltpu.sync_copy(edst_hbm.at[wid], idx_d)

    def start(c, buf, sem):
        pltpu.async_copy(x_hbm.at[idx_s.at[c]], buf, sem)

    def wait(buf, sem):
        pltpu.make_async_copy(x_hbm.at[idx_s.at[0]], buf, sem).wait()

    def scat(c, buf):
        pltpu.sync_copy(buf, feat_sh.at[idx_d.at[c]], add=True)
        pltpu.async_copy(ones_v, deg_sh.at[idx_d.at[c]], sem_d, add=True)

    def wait_deg():
        pltpu.make_async_copy(ones_v, deg_sh.at[idx_d.at[0]], sem_d).wait()

    start(0, rows0, sem0)

    @pl.loop(0, NCH // 2)
    def _(j):
        c0 = 2 * j
        start(c0 + 1, rows1, sem1)
        wait(rows0, sem0)
        scat(c0, rows0)
        start(c0 + 2, rows0, sem0)
        wait(rows1, sem1)
        scat(c0 + 1, rows1)

        @pl.when(j >= 2)
        def _():
            wait_deg()
            wait_deg()

    wait(rows0, sem0)
    scat(NCH - 1, rows0)
    for _ in range(5):
        wait_deg()

    plsc.subcore_barrier()
    pltpu.sync_copy(feat_sh.at[pl.ds(row0, ROWS_PER_TILE)],
                    feat_hbm.at[cid, pl.ds(row0, ROWS_PER_TILE)])
    pltpu.sync_copy(deg_sh.at[pl.ds(row0, ROWS_PER_TILE)],
                    deg_hbm.at[cid, pl.ds(row0, ROWS_PER_TILE)])


_agg_call = pl.kernel(
    _agg_body,
    out_type=[
        jax.ShapeDtypeStruct((NC, N_PAD, D), jnp.float32),
        jax.ShapeDtypeStruct((NC, N_PAD), jnp.float32),
    ],
    mesh=_mesh,
    scratch_types=[
        pltpu.VMEM_SHARED((N_PAD, D), jnp.float32),
        pltpu.VMEM_SHARED((N_PAD,), jnp.float32),
        pltpu.VMEM((NCH, CH), jnp.int32),
        pltpu.VMEM((NCH, CH), jnp.int32),
        pltpu.VMEM((CH, D), jnp.float32),
        pltpu.VMEM((CH, D), jnp.float32),
        pltpu.VMEM((ROWS_PER_TILE,), jnp.float32),
        pltpu.VMEM((CH,), jnp.float32),
        pltpu.SemaphoreType.DMA,
        pltpu.SemaphoreType.DMA,
        pltpu.SemaphoreType.DMA,
    ],
    compiler_params=_sc_params,
)


RB = 1000


def _densea_body(x_ref, ws_ref, xs_ref):
    xs_ref[...] = jnp.dot(x_ref[...], ws_ref[...],
                          preferred_element_type=jnp.float32)


_densea_call = pl.pallas_call(
    _densea_body,
    grid=(N_NODES // RB,),
    in_specs=[
        pl.BlockSpec((RB, D), lambda i: (i, 0)),
        pl.BlockSpec((D, D), lambda i: (0, 0)),
    ],
    out_specs=pl.BlockSpec((RB, D), lambda i: (i, 0)),
    out_shape=jax.ShapeDtypeStruct((N_NODES, D), jnp.float32),
)


def _denseb_body(feat_ref, deg_ref, xs_ref, wn_ref, h_ref):
    a = feat_ref[0] + feat_ref[1]
    deg = deg_ref[:, 0:1] + deg_ref[:, 1:2]
    mean = a / jnp.maximum(deg, 1.0)
    h_ref[...] = jnp.maximum(
        xs_ref[...]
        + jnp.dot(mean, wn_ref[...], preferred_element_type=jnp.float32),
        0.0)


_denseb_call = pl.pallas_call(
    _denseb_body,
    grid=(N_NODES // RB,),
    in_specs=[
        pl.BlockSpec((NC, RB, D), lambda i: (0, i, 0)),
        pl.BlockSpec((RB, NC), lambda i: (i, 0)),
        pl.BlockSpec((RB, D), lambda i: (i, 0)),
        pl.BlockSpec((D, D), lambda i: (0, 0)),
    ],
    out_specs=pl.BlockSpec((RB, D), lambda i: (i, 0)),
    out_shape=jax.ShapeDtypeStruct((N_NODES, D), jnp.float32),
)


def _pairgather_body(h_hbm, src_hbm, dst_hbm, nb_hbm,
                     us_hbm, vd_hbm, bsum_hbm,
                     idx_s, idx_d, hs, hd, nb_v, bias_v, sem0, sem1):
    cid = lax.axis_index("c")
    sid = lax.axis_index("s")
    wid = cid * NS + sid

    pltpu.sync_copy(src_hbm.at[wid], idx_s)
    pltpu.sync_copy(dst_hbm.at[wid], idx_d)
    pltpu.sync_copy(nb_hbm, nb_v)

    @pl.loop(0, NSCH)
    def _(c):
        cp0 = pltpu.async_copy(h_hbm.at[idx_s.at[c]], hs, sem0)
        cp1 = pltpu.async_copy(h_hbm.at[idx_d.at[c]], hd, sem1)
        for g in range(SCH // 16):
            sv = idx_s[c, pl.ds(16 * g, 16)]
            dv = idx_d[c, pl.ds(16 * g, 16)]
            bs = plsc.load_gather(nb_v, [sv])
            bd = plsc.load_gather(nb_v, [dv])
            bias_v[pl.ds(16 * g, 16)] = bs + bd
        cp0.wait()
        cp1.wait()
        base = wid * PAIRS_PER_W + c * SCH
        pltpu.sync_copy(hs, us_hbm.at[pl.ds(base, SCH)])
        pltpu.sync_copy(hd, vd_hbm.at[pl.ds(base, SCH)])
        pltpu.sync_copy(bias_v, bsum_hbm.at[pl.ds(base, SCH)])


_pairgather_call = pl.kernel(
    _pairgather_body,
    out_type=[
        jax.ShapeDtypeStruct((B, D), jnp.float32),
        jax.ShapeDtypeStruct((B, D), jnp.float32),
        jax.ShapeDtypeStruct((B,), jnp.float32),
    ],
    mesh=_mesh,
    scratch_types=[
        pltpu.VMEM((NSCH, SCH), jnp.int32),
        pltpu.VMEM((NSCH, SCH), jnp.int32),
        pltpu.VMEM((SCH, D), jnp.float32),
        pltpu.VMEM((SCH, D), jnp.float32),
        pltpu.VMEM((N_NODES,), jnp.float32),
        pltpu.VMEM((SCH,), jnp.float32),
        pltpu.SemaphoreType.DMA,
        pltpu.SemaphoreType.DMA,
    ],
    compiler_params=_sc_params_nl,
)


SB = 2048


def _score_body(us_ref, vd_ref, bsum_ref, out_ref):
    out_ref[...] = jnp.sum(us_ref[...] * vd_ref[...], axis=1) + bsum_ref[...]


_score_call = pl.pallas_call(
    _score_body,
    grid=(B // SB,),
    in_specs=[
        pl.BlockSpec((SB, D), lambda i: (i, 0)),
        pl.BlockSpec((SB, D), lambda i: (i, 0)),
        pl.BlockSpec((SB,), lambda i: (i,)),
    ],
    out_specs=pl.BlockSpec((SB,), lambda i: (i,)),
    out_shape=jax.ShapeDtypeStruct((B,), jnp.float32),
)


def kernel(x, edge_index, src, dst, W_self, W_neigh, node_biases):
    x = x.astype(jnp.float32)
    e_src = edge_index[0].astype(jnp.int32).reshape(NW, NCH, CH)
    e_dst = edge_index[1].astype(jnp.int32).reshape(NW, NCH, CH)
    src3 = src.astype(jnp.int32).reshape(NW, NSCH, SCH)
    dst3 = dst.astype(jnp.int32).reshape(NW, NSCH, SCH)
    nb = node_biases[1:N_NODES + 1].astype(jnp.float32)

    feat, deg = _agg_call(x, e_src, e_dst)
    xs = _densea_call(x, W_self)
    h = _denseb_call(feat, deg.T, xs, W_neigh)
    us, vd, bsum = _pairgather_call(h, src3, dst3, nb)
    return _score_call(us, vd, bsum)
